# baseline (speedup 1.0000x reference)
import functools

import jax
import jax.numpy as jnp
from jax import lax
from jax.experimental import pallas as pl
from jax.experimental.pallas import tpu as pltpu
from jax.experimental.pallas import tpu_sc as plsc

_NC = 2
_NS = 16
_C = 160
_CT = 80


def _sc_segment_sum(t0, t1, src, dst, n_pad, with_deg):
    n_rows, f_half = t0.shape
    e_total = src.shape[0]
    ept = e_total // _NS
    tails = (_C, _C, _CT)
    n_main = (ept - sum(tails)) // _C
    assert n_main * _C + sum(tails) == ept and n_main % 4 == 0
    rpt = n_pad // _NS
    z2 = jnp.zeros((rpt, f_half), jnp.float32)
    z1 = jnp.zeros((rpt,), jnp.float32)
    ones_c = jnp.ones((_C,), jnp.float32)

    mesh = plsc.VectorSubcoreMesh(core_axis_name="c", subcore_axis_name="s",
                                  num_cores=_NC, num_subcores=_NS)

    out_type = [
        jax.ShapeDtypeStruct((n_pad, f_half), jnp.float32),
        jax.ShapeDtypeStruct((n_pad, f_half), jnp.float32),
    ]
    scratch = (
        [pltpu.VMEM((_C,), jnp.int32)] * 8
        + [pltpu.VMEM((_C, f_half), jnp.float32)] * 2
        + [
            pltpu.VMEM((_CT,), jnp.int32),
            pltpu.VMEM((_CT,), jnp.int32),
            pltpu.VMEM((_C,), jnp.float32),
            pltpu.VMEM_SHARED((n_pad, f_half), jnp.float32),
            pltpu.VMEM_SHARED((n_pad,), jnp.float32),
        ]
        + [pltpu.SemaphoreType.DMA] * 10
    )
    if with_deg:
        out_type.append(jax.ShapeDtypeStruct((n_pad,), jnp.float32))

    @functools.partial(pl.kernel, out_type=tuple(out_type), mesh=mesh,
                       scratch_types=scratch)
    def seg_kernel(t0_hbm, t1_hbm, src_hbm, dst_hbm, z2_hbm, z1_hbm, ones_hbm,
                   agg0_hbm, agg1_hbm, *rest):
        if with_deg:
            deg_hbm = rest[0]
            rest = rest[1:]
        (si0, di0, si1, di1, si2, di2, si3, di3, rows_v0, rows_v1,
         src_vt, dst_vt, ones_v, acc_sh, deg_sh,
         gsem0, gsem1, isem0, isem1, isem2, isem3,
         ssem0, ssem1, dsem0, dsem1) = rest
        idx = ((si0, di0, isem0), (si1, di1, isem1),
               (si2, di2, isem2), (si3, di3, isem3))
        rows = ((rows_v0, gsem0), (rows_v1, gsem1))
        ssem = (ssem0, ssem1)
        dsem = (dsem0, dsem1)
        cid = lax.axis_index("c")
        sid = lax.axis_index("s")
        row0 = pl.multiple_of(sid * rpt, 8)
        ebase = pl.multiple_of(sid * ept, 8)

        pltpu.sync_copy(z2_hbm, acc_sh.at[pl.ds(row0, rpt)])

        if with_deg:
            @pl.when(cid == 0)
            def _():
                pltpu.sync_copy(z1_hbm, deg_sh.at[pl.ds(row0, rpt)])

        pltpu.sync_copy(ones_hbm, ones_v)
        plsc.subcore_barrier()

        def idx_off(k):
            return pl.multiple_of(ebase + k * _C, 8)

        def fire_idx(k, s):
            src_s, dst_s, isem_s = idx[s]
            off = idx_off(k)
            pltpu.async_copy(src_hbm.at[pl.ds(off, _C)], src_s, isem_s)
            pltpu.async_copy(dst_hbm.at[pl.ds(off, _C)], dst_s, isem_s)

        def wait_idx(k, s):
            src_s, dst_s, isem_s = idx[s]
            off = idx_off(k)
            pltpu.make_async_copy(src_hbm.at[pl.ds(off, _C)], src_s,
                                  isem_s).wait()
            pltpu.make_async_copy(dst_hbm.at[pl.ds(off, _C)], dst_s,
                                  isem_s).wait()

        def edge_loop(t_hbm, do_deg):
            for k in range(3):
                fire_idx(k, k)
            wait_idx(0, 0)
            pltpu.async_copy(t_hbm.at[idx[0][0]], rows[0][0], rows[0][1])

            def outer(g, carry):
                for j in range(4):
                    k = 4 * g + j
                    b = j % 2
                    rows_b, gsem_b = rows[b]
                    rows_p, _ = rows[1 - b]
                    dst_j = idx[j][1]
                    dst_p = idx[(j + 3) % 4][1]

                    pltpu.make_async_copy(t_hbm.at[idx[j][0]], rows_b,
                                          gsem_b).wait()
                    pltpu.async_copy(rows_b, acc_sh.at[dst_j], ssem[b],
                                     add=True)
                    if do_deg:
                        pltpu.async_copy(ones_v, deg_sh.at[dst_j], dsem[b],
                                         add=True)

                    @pl.when(k >= 1)
                    def _():
                        pltpu.make_async_copy(rows_p, acc_sh.at[dst_p],
                                              ssem[1 - b]).wait()
                        if do_deg:
                            pltpu.make_async_copy(ones_v, deg_sh.at[dst_p],
                                                  dsem[1 - b]).wait()

                    @pl.when(k + 3 < n_main)
                    def _():
                        fire_idx(k + 3, (j + 3) % 4)

                    @pl.when(k + 1 < n_main)
                    def _():
                        wait_idx(k + 1, (j + 1) % 4)
                        pltpu.async_copy(t_hbm.at[idx[(j + 1) % 4][0]],
                                         rows_p, rows[1 - b][1])
                return carry

            lax.fori_loop(0, n_main // 4, outer, 0)

            lb = (n_main - 1) % 2
            lj = (n_main - 1) % 4
            pltpu.make_async_copy(rows[lb][0], acc_sh.at[idx[lj][1]],
                                  ssem[lb]).wait()
            if do_deg:
                pltpu.make_async_copy(ones_v, deg_sh.at[idx[lj][1]],
                                      dsem[lb]).wait()

            off = n_main * _C
            for tc in tails:
                src_t = src_vt if tc == _CT else si0
                dst_t = dst_vt if tc == _CT else di0
                o = pl.multiple_of(ebase + off, 8)
                pltpu.sync_copy(src_hbm.at[pl.ds(o, tc)], src_t)
                pltpu.sync_copy(dst_hbm.at[pl.ds(o, tc)], dst_t)
                rows_t = rows_v0.at[pl.ds(0, tc)] if tc != _C else rows_v0
                pltpu.async_copy(t_hbm.at[src_t], rows_t, gsem0).wait()
                pltpu.sync_copy(rows_t, acc_sh.at[dst_t], add=True)
                if do_deg:
                    ones_t = ones_v.at[pl.ds(0, tc)] if tc != _C else ones_v
                    pltpu.sync_copy(ones_t, deg_sh.at[dst_t], add=True)
                off += tc

        @pl.when(cid == 0)
        def _():
            edge_loop(t0_hbm, with_deg)

        @pl.when(cid == 1)
        def _():
            edge_loop(t1_hbm, False)

        plsc.subcore_barrier()

        @pl.when(cid == 0)
        def _():
            pltpu.sync_copy(acc_sh.at[pl.ds(row0, rpt)],
                            agg0_hbm.at[pl.ds(row0, rpt)])
            if with_deg:
                pltpu.sync_copy(deg_sh.at[pl.ds(row0, rpt)],
                                deg_hbm.at[pl.ds(row0, rpt)])

        @pl.when(cid == 1)
        def _():
            pltpu.sync_copy(acc_sh.at[pl.ds(row0, rpt)],
                            agg1_hbm.at[pl.ds(row0, rpt)])

    return seg_kernel(t0, t1, src, dst, z2, z1, ones_c)


def _dotT(a, w):
    return lax.dot_general(a, w, (((1,), (1,)), ((), ())),
                           preferred_element_type=jnp.float32)


def _tc_matmul_bias(a, w, b, block_rows):
    n_rows, d_k = a.shape
    d_out = w.shape[0]
    grid = n_rows // block_rows

    def body(ar, wr, br, out_ref):
        out_ref[...] = _dotT(ar[...], wr[...]) + br[...]

    return pl.pallas_call(
        body,
        grid=(grid,),
        in_specs=[
            pl.BlockSpec((block_rows, d_k), lambda i: (i, 0)),
            pl.BlockSpec((d_out, d_k), lambda i: (0, 0)),
            pl.BlockSpec((1, d_out), lambda i: (0, 0)),
        ],
        out_specs=pl.BlockSpec((block_rows, d_out), lambda i: (i, 0)),
        out_shape=jax.ShapeDtypeStruct((n_rows, d_out), jnp.float32),
    )(a, w, b)


def _tc_h_p(agg0, agg1, deg2, q, W1_l, W2_l, block_rows):
    n_rows, d_h = q.shape
    f_half = agg0.shape[1]
    grid = n_rows // block_rows

    def body(a0, a1, dg, qr, wl, w2, h_ref, p0_ref, p1_ref):
        scale = 1.0 / jnp.maximum(dg[...], 1.0)
        wl_full = wl[...]
        s = _dotT(a0[...], wl_full[:, :f_half])
        s = s + _dotT(a1[...], wl_full[:, f_half:])
        h = jnp.maximum(s * scale + qr[...], 0.0)
        h_ref[...] = h
        p = _dotT(h, w2[...])
        p0_ref[...] = p[:, :f_half]
        p1_ref[...] = p[:, f_half:]

    return pl.pallas_call(
        body,
        grid=(grid,),
        in_specs=[
            pl.BlockSpec((block_rows, f_half), lambda i: (i, 0)),
            pl.BlockSpec((block_rows, f_half), lambda i: (i, 0)),
            pl.BlockSpec((block_rows, 1), lambda i: (i, 0)),
            pl.BlockSpec((block_rows, d_h), lambda i: (i, 0)),
            pl.BlockSpec((d_h, 2 * f_half), lambda i: (0, 0)),
            pl.BlockSpec((2 * f_half, d_h), lambda i: (0, 0)),
        ],
        out_specs=[
            pl.BlockSpec((block_rows, d_h), lambda i: (i, 0)),
            pl.BlockSpec((block_rows, f_half), lambda i: (i, 0)),
            pl.BlockSpec((block_rows, f_half), lambda i: (i, 0)),
        ],
        out_shape=[
            jax.ShapeDtypeStruct((n_rows, d_h), jnp.float32),
            jax.ShapeDtypeStruct((n_rows, f_half), jnp.float32),
            jax.ShapeDtypeStruct((n_rows, f_half), jnp.float32),
        ],
    )(agg0, agg1, deg2, q, W1_l, W2_l)


def _tc_combine(agg0, agg1, deg2, r, block_rows):
    n_rows, d_out = r.shape
    f_half = agg0.shape[1]
    grid = n_rows // block_rows

    def body(a0, a1, dg, rr, out_ref):
        scale = 1.0 / jnp.maximum(dg[...], 1.0)
        rf = rr[...]
        out_ref[:, :f_half] = a0[...] * scale + rf[:, :f_half]
        out_ref[:, f_half:] = a1[...] * scale + rf[:, f_half:]

    return pl.pallas_call(
        body,
        grid=(grid,),
        in_specs=[
            pl.BlockSpec((block_rows, f_half), lambda i: (i, 0)),
            pl.BlockSpec((block_rows, f_half), lambda i: (i, 0)),
            pl.BlockSpec((block_rows, 1), lambda i: (i, 0)),
            pl.BlockSpec((block_rows, d_out), lambda i: (i, 0)),
        ],
        out_specs=pl.BlockSpec((block_rows, d_out), lambda i: (i, 0)),
        out_shape=jax.ShapeDtypeStruct((n_rows, d_out), jnp.float32),
    )(agg0, agg1, deg2, r)


def kernel(x, edge_index, W1_l, b1, W1_r, W2_l, b2, W2_r):
    n, d_in = x.shape
    f_half = d_in // 2
    src = edge_index[0]
    dst = edge_index[1]
    n_pad = ((n + 128 * _NS - 1) // (128 * _NS)) * (128 * _NS)

    x0 = x[:, :f_half]
    x1 = x[:, f_half:]
    q = _tc_matmul_bias(x, W1_r, b1.reshape(1, -1), block_rows=2000)
    agg10, agg11, deg = _sc_segment_sum(x0, x1, src, dst, n_pad, True)
    deg2 = deg[:n].reshape(n, 1)
    h, p0, p1 = _tc_h_p(agg10[:n], agg11[:n], deg2, q, W1_l, W2_l,
                        block_rows=2000)
    agg20, agg21 = _sc_segment_sum(p0, p1, src, dst, n_pad, False)
    r = _tc_matmul_bias(h, W2_r, b2.reshape(1, -1), block_rows=2000)
    out = _tc_combine(agg20[:n], agg21[:n], deg2, r, block_rows=2000)
    return out


# speedup vs baseline: 1.0836x; 1.0836x over previous
import functools

import jax
import jax.numpy as jnp
from jax import lax
from jax.experimental import pallas as pl
from jax.experimental.pallas import tpu as pltpu
from jax.experimental.pallas import tpu_sc as plsc

_NC = 2
_NS = 16
_C = 160
_CT = 80


def _sc_segment_sum(t0, t1, src, dst, n_pad, with_deg):
    n_rows, f_half = t0.shape
    e_total = src.shape[0]
    ept = e_total // _NS
    tails = (_C, _C, _CT)
    n_main = (ept - sum(tails)) // _C
    assert n_main * _C + sum(tails) == ept and n_main % 4 == 0
    rpt = n_pad // _NS
    z2 = jnp.zeros((rpt, f_half), jnp.float32)
    z1 = jnp.zeros((rpt,), jnp.float32)
    ones_c = jnp.ones((_C,), jnp.float32)

    mesh = plsc.VectorSubcoreMesh(core_axis_name="c", subcore_axis_name="s",
                                  num_cores=_NC, num_subcores=_NS)

    out_type = [
        jax.ShapeDtypeStruct((n_pad, f_half), jnp.float32),
        jax.ShapeDtypeStruct((n_pad, f_half), jnp.float32),
    ]
    scratch = (
        [pltpu.VMEM((_C,), jnp.int32)] * 8
        + [pltpu.VMEM((_C, f_half), jnp.float32)] * 2
        + [
            pltpu.VMEM((_CT,), jnp.int32),
            pltpu.VMEM((_CT,), jnp.int32),
            pltpu.VMEM((_C,), jnp.float32),
            pltpu.VMEM_SHARED((n_pad, f_half), jnp.float32),
            pltpu.VMEM_SHARED((n_pad,), jnp.float32),
        ]
        + [pltpu.SemaphoreType.DMA] * 6
    )
    if with_deg:
        out_type.append(jax.ShapeDtypeStruct((n_pad,), jnp.float32))

    @functools.partial(pl.kernel, out_type=tuple(out_type), mesh=mesh,
                       scratch_types=scratch)
    def seg_kernel(t0_hbm, t1_hbm, src_hbm, dst_hbm, z2_hbm, z1_hbm, ones_hbm,
                   agg0_hbm, agg1_hbm, *rest):
        if with_deg:
            deg_hbm = rest[0]
            rest = rest[1:]
        (si0, di0, si1, di1, si2, di2, si3, di3, rows_v0, rows_v1,
         src_vt, dst_vt, ones_v, acc_sh, deg_sh,
         gsem0, gsem1, isem0, isem1, isem2, isem3) = rest
        idx = ((si0, di0, isem0), (si1, di1, isem1),
               (si2, di2, isem2), (si3, di3, isem3))
        rows = ((rows_v0, gsem0), (rows_v1, gsem1))
        cid = lax.axis_index("c")
        sid = lax.axis_index("s")
        row0 = pl.multiple_of(sid * rpt, 8)
        ebase = pl.multiple_of(sid * ept, 8)

        pltpu.sync_copy(z2_hbm, acc_sh.at[pl.ds(row0, rpt)])

        if with_deg:
            @pl.when(cid == 0)
            def _():
                pltpu.sync_copy(z1_hbm, deg_sh.at[pl.ds(row0, rpt)])

        pltpu.sync_copy(ones_hbm, ones_v)
        plsc.subcore_barrier()

        def idx_off(k):
            return pl.multiple_of(ebase + k * _C, 8)

        def fire_idx(k, s):
            src_s, dst_s, isem_s = idx[s]
            off = idx_off(k)
            pltpu.async_copy(src_hbm.at[pl.ds(off, _C)], src_s, isem_s)
            pltpu.async_copy(dst_hbm.at[pl.ds(off, _C)], dst_s, isem_s)

        def wait_idx(k, s):
            src_s, dst_s, isem_s = idx[s]
            off = idx_off(k)
            pltpu.make_async_copy(src_hbm.at[pl.ds(off, _C)], src_s,
                                  isem_s).wait()
            pltpu.make_async_copy(dst_hbm.at[pl.ds(off, _C)], dst_s,
                                  isem_s).wait()

        def edge_loop(t_hbm, do_deg):
            for k in range(3):
                fire_idx(k, k)
            for k in range(2):
                wait_idx(k, k)
                pltpu.async_copy(t_hbm.at[idx[k][0]], rows[k][0], rows[k][1])

            def outer(g, carry):
                for j in range(4):
                    k = 4 * g + j
                    rows_b, gsem_b = rows[j % 2]
                    dst_j = idx[j][1]

                    @pl.when(k + 3 < n_main)
                    def _():
                        fire_idx(k + 3, (j + 3) % 4)

                    pltpu.make_async_copy(t_hbm.at[idx[j][0]], rows_b,
                                          gsem_b).wait()
                    pltpu.sync_copy(rows_b, acc_sh.at[dst_j], add=True)
                    if do_deg:
                        pltpu.sync_copy(ones_v, deg_sh.at[dst_j], add=True)

                    @pl.when(k + 2 < n_main)
                    def _():
                        wait_idx(k + 2, (j + 2) % 4)
                        pltpu.async_copy(t_hbm.at[idx[(j + 2) % 4][0]],
                                         rows_b, gsem_b)
                return carry

            lax.fori_loop(0, n_main // 4, outer, 0)

            off = n_main * _C
            for tc in tails:
                src_t = src_vt if tc == _CT else si0
                dst_t = dst_vt if tc == _CT else di0
                o = pl.multiple_of(ebase + off, 8)
                pltpu.sync_copy(src_hbm.at[pl.ds(o, tc)], src_t)
                pltpu.sync_copy(dst_hbm.at[pl.ds(o, tc)], dst_t)
                rows_t = rows_v0.at[pl.ds(0, tc)] if tc != _C else rows_v0
                pltpu.async_copy(t_hbm.at[src_t], rows_t, gsem0).wait()
                pltpu.sync_copy(rows_t, acc_sh.at[dst_t], add=True)
                if do_deg:
                    ones_t = ones_v.at[pl.ds(0, tc)] if tc != _C else ones_v
                    pltpu.sync_copy(ones_t, deg_sh.at[dst_t], add=True)
                off += tc

        @pl.when(cid == 0)
        def _():
            edge_loop(t0_hbm, with_deg)

        @pl.when(cid == 1)
        def _():
            edge_loop(t1_hbm, False)

        plsc.subcore_barrier()

        @pl.when(cid == 0)
        def _():
            pltpu.sync_copy(acc_sh.at[pl.ds(row0, rpt)],
                            agg0_hbm.at[pl.ds(row0, rpt)])
            if with_deg:
                pltpu.sync_copy(deg_sh.at[pl.ds(row0, rpt)],
                                deg_hbm.at[pl.ds(row0, rpt)])

        @pl.when(cid == 1)
        def _():
            pltpu.sync_copy(acc_sh.at[pl.ds(row0, rpt)],
                            agg1_hbm.at[pl.ds(row0, rpt)])

    return seg_kernel(t0, t1, src, dst, z2, z1, ones_c)


def _dotT(a, w):
    return lax.dot_general(a.astype(jnp.bfloat16), w.astype(jnp.bfloat16),
                           (((1,), (1,)), ((), ())),
                           preferred_element_type=jnp.float32)


def _tc_matmul_bias(a, w, b, block_rows):
    n_rows, d_k = a.shape
    d_out = w.shape[0]
    grid = n_rows // block_rows

    def body(ar, wr, br, out_ref):
        out_ref[...] = _dotT(ar[...], wr[...]) + br[...]

    return pl.pallas_call(
        body,
        grid=(grid,),
        in_specs=[
            pl.BlockSpec((block_rows, d_k), lambda i: (i, 0)),
            pl.BlockSpec((d_out, d_k), lambda i: (0, 0)),
            pl.BlockSpec((1, d_out), lambda i: (0, 0)),
        ],
        out_specs=pl.BlockSpec((block_rows, d_out), lambda i: (i, 0)),
        out_shape=jax.ShapeDtypeStruct((n_rows, d_out), jnp.float32),
    )(a, w, b)


def _tc_h_p(agg0, agg1, deg2, q, W1_l, W2_l, block_rows):
    n_rows, d_h = q.shape
    f_half = agg0.shape[1]
    grid = n_rows // block_rows

    def body(a0, a1, dg, qr, wl, w2, h_ref, p0_ref, p1_ref):
        scale = 1.0 / jnp.maximum(dg[...], 1.0)
        wl_full = wl[...]
        s = _dotT(a0[...], wl_full[:, :f_half])
        s = s + _dotT(a1[...], wl_full[:, f_half:])
        h = jnp.maximum(s * scale + qr[...], 0.0)
        h_ref[...] = h
        p = _dotT(h, w2[...])
        p0_ref[...] = p[:, :f_half]
        p1_ref[...] = p[:, f_half:]

    return pl.pallas_call(
        body,
        grid=(grid,),
        in_specs=[
            pl.BlockSpec((block_rows, f_half), lambda i: (i, 0)),
            pl.BlockSpec((block_rows, f_half), lambda i: (i, 0)),
            pl.BlockSpec((block_rows, 1), lambda i: (i, 0)),
            pl.BlockSpec((block_rows, d_h), lambda i: (i, 0)),
            pl.BlockSpec((d_h, 2 * f_half), lambda i: (0, 0)),
            pl.BlockSpec((2 * f_half, d_h), lambda i: (0, 0)),
        ],
        out_specs=[
            pl.BlockSpec((block_rows, d_h), lambda i: (i, 0)),
            pl.BlockSpec((block_rows, f_half), lambda i: (i, 0)),
            pl.BlockSpec((block_rows, f_half), lambda i: (i, 0)),
        ],
        out_shape=[
            jax.ShapeDtypeStruct((n_rows, d_h), jnp.float32),
            jax.ShapeDtypeStruct((n_rows, f_half), jnp.float32),
            jax.ShapeDtypeStruct((n_rows, f_half), jnp.float32),
        ],
    )(agg0, agg1, deg2, q, W1_l, W2_l)


def _tc_combine(agg0, agg1, deg2, r, block_rows):
    n_rows, d_out = r.shape
    f_half = agg0.shape[1]
    grid = n_rows // block_rows

    def body(a0, a1, dg, rr, out_ref):
        scale = 1.0 / jnp.maximum(dg[...], 1.0)
        rf = rr[...]
        out_ref[:, :f_half] = a0[...] * scale + rf[:, :f_half]
        out_ref[:, f_half:] = a1[...] * scale + rf[:, f_half:]

    return pl.pallas_call(
        body,
        grid=(grid,),
        in_specs=[
            pl.BlockSpec((block_rows, f_half), lambda i: (i, 0)),
            pl.BlockSpec((block_rows, f_half), lambda i: (i, 0)),
            pl.BlockSpec((block_rows, 1), lambda i: (i, 0)),
            pl.BlockSpec((block_rows, d_out), lambda i: (i, 0)),
        ],
        out_specs=pl.BlockSpec((block_rows, d_out), lambda i: (i, 0)),
        out_shape=jax.ShapeDtypeStruct((n_rows, d_out), jnp.float32),
    )(agg0, agg1, deg2, r)


def kernel(x, edge_index, W1_l, b1, W1_r, W2_l, b2, W2_r):
    n, d_in = x.shape
    f_half = d_in // 2
    src = edge_index[0]
    dst = edge_index[1]
    n_pad = ((n + 128 * _NS - 1) // (128 * _NS)) * (128 * _NS)

    x0 = x[:, :f_half]
    x1 = x[:, f_half:]
    q = _tc_matmul_bias(x, W1_r, b1.reshape(1, -1), block_rows=2000)
    agg10, agg11, deg = _sc_segment_sum(x0, x1, src, dst, n_pad, True)
    deg2 = deg[:n].reshape(n, 1)
    h, p0, p1 = _tc_h_p(agg10[:n], agg11[:n], deg2, q, W1_l, W2_l,
                        block_rows=2000)
    agg20, agg21 = _sc_segment_sum(p0, p1, src, dst, n_pad, False)
    r = _tc_matmul_bias(h, W2_r, b2.reshape(1, -1), block_rows=2000)
    out = _tc_combine(agg20[:n], agg21[:n], deg2, r, block_rows=2000)
    return out


# speedup vs baseline: 1.1194x; 1.0330x over previous
import functools

import jax
import jax.numpy as jnp
from jax import lax
from jax.experimental import pallas as pl
from jax.experimental.pallas import tpu as pltpu
from jax.experimental.pallas import tpu_sc as plsc

_NC = 2
_NS = 16
_C = 160
_CT = 80


def _sc_segment_sum(t0, t1, src, dst, n_pad, with_deg):
    n_rows, f_half = t0.shape
    e_total = src.shape[0]
    ept = e_total // _NS
    tails = (_C, _C, _CT)
    n_main = (ept - sum(tails)) // _C
    assert n_main * _C + sum(tails) == ept and n_main % 4 == 0
    rpt = n_pad // _NS
    z2 = jnp.zeros((rpt, f_half), jnp.float32)
    z1 = jnp.zeros((rpt,), jnp.float32)
    ones_c = jnp.ones((_C,), jnp.float32)

    mesh = plsc.VectorSubcoreMesh(core_axis_name="c", subcore_axis_name="s",
                                  num_cores=_NC, num_subcores=_NS)

    out_type = [
        jax.ShapeDtypeStruct((n_pad, f_half), jnp.float32),
        jax.ShapeDtypeStruct((n_pad, f_half), jnp.float32),
    ]
    scratch = (
        [pltpu.VMEM((_C,), jnp.int32)] * 8
        + [pltpu.VMEM((_C, f_half), jnp.float32)] * 2
        + [
            pltpu.VMEM((_CT,), jnp.int32),
            pltpu.VMEM((_CT,), jnp.int32),
            pltpu.VMEM((_C,), jnp.float32),
            pltpu.VMEM_SHARED((n_pad, f_half), jnp.float32),
            pltpu.VMEM_SHARED((n_pad,), jnp.float32),
        ]
        + [pltpu.SemaphoreType.DMA] * 6
    )
    if with_deg:
        out_type.append(jax.ShapeDtypeStruct((n_pad,), jnp.float32))

    @functools.partial(pl.kernel, out_type=tuple(out_type), mesh=mesh,
                       scratch_types=scratch)
    def seg_kernel(t0_hbm, t1_hbm, src_hbm, dst_hbm, z2_hbm, z1_hbm, ones_hbm,
                   agg0_hbm, agg1_hbm, *rest):
        if with_deg:
            deg_hbm = rest[0]
            rest = rest[1:]
        (si0, di0, si1, di1, si2, di2, si3, di3, rows_v0, rows_v1,
         src_vt, dst_vt, ones_v, acc_sh, deg_sh,
         gsem0, gsem1, isem0, isem1, isem2, isem3) = rest
        idx = ((si0, di0, isem0), (si1, di1, isem1),
               (si2, di2, isem2), (si3, di3, isem3))
        rows = ((rows_v0, gsem0), (rows_v1, gsem1))
        cid = lax.axis_index("c")
        sid = lax.axis_index("s")
        row0 = pl.multiple_of(sid * rpt, 8)
        ebase = pl.multiple_of(sid * ept, 8)

        pltpu.sync_copy(z2_hbm, acc_sh.at[pl.ds(row0, rpt)])

        if with_deg:
            @pl.when(cid == 0)
            def _():
                pltpu.sync_copy(z1_hbm, deg_sh.at[pl.ds(row0, rpt)])

        pltpu.sync_copy(ones_hbm, ones_v)
        plsc.subcore_barrier()

        def idx_off(k):
            return pl.multiple_of(ebase + k * _C, 8)

        def fire_idx(k, s):
            src_s, dst_s, isem_s = idx[s]
            off = idx_off(k)
            pltpu.async_copy(src_hbm.at[pl.ds(off, _C)], src_s, isem_s)
            pltpu.async_copy(dst_hbm.at[pl.ds(off, _C)], dst_s, isem_s)

        def wait_idx(k, s):
            src_s, dst_s, isem_s = idx[s]
            off = idx_off(k)
            pltpu.make_async_copy(src_hbm.at[pl.ds(off, _C)], src_s,
                                  isem_s).wait()
            pltpu.make_async_copy(dst_hbm.at[pl.ds(off, _C)], dst_s,
                                  isem_s).wait()

        def edge_loop(t_hbm, do_deg):
            for k in range(3):
                fire_idx(k, k)
            for k in range(2):
                wait_idx(k, k)
                pltpu.async_copy(t_hbm.at[idx[k][0]], rows[k][0], rows[k][1])

            def outer(g, carry):
                for j in range(4):
                    k = 4 * g + j
                    rows_b, gsem_b = rows[j % 2]
                    dst_j = idx[j][1]

                    @pl.when(k + 3 < n_main)
                    def _():
                        fire_idx(k + 3, (j + 3) % 4)

                    pltpu.make_async_copy(t_hbm.at[idx[j][0]], rows_b,
                                          gsem_b).wait()
                    pltpu.sync_copy(rows_b, acc_sh.at[dst_j], add=True)
                    if do_deg:
                        pltpu.sync_copy(ones_v, deg_sh.at[dst_j], add=True)

                    @pl.when(k + 2 < n_main)
                    def _():
                        wait_idx(k + 2, (j + 2) % 4)
                        pltpu.async_copy(t_hbm.at[idx[(j + 2) % 4][0]],
                                         rows_b, gsem_b)
                return carry

            lax.fori_loop(0, n_main // 4, outer, 0)

            off = n_main * _C
            for tc in tails:
                src_t = src_vt if tc == _CT else si0
                dst_t = dst_vt if tc == _CT else di0
                o = pl.multiple_of(ebase + off, 8)
                pltpu.sync_copy(src_hbm.at[pl.ds(o, tc)], src_t)
                pltpu.sync_copy(dst_hbm.at[pl.ds(o, tc)], dst_t)
                rows_t = rows_v0.at[pl.ds(0, tc)] if tc != _C else rows_v0
                pltpu.async_copy(t_hbm.at[src_t], rows_t, gsem0).wait()
                pltpu.sync_copy(rows_t, acc_sh.at[dst_t], add=True)
                if do_deg:
                    ones_t = ones_v.at[pl.ds(0, tc)] if tc != _C else ones_v
                    pltpu.sync_copy(ones_t, deg_sh.at[dst_t], add=True)
                off += tc

        @pl.when(cid == 0)
        def _():
            edge_loop(t0_hbm, with_deg)

        @pl.when(cid == 1)
        def _():
            edge_loop(t1_hbm, False)

        plsc.subcore_barrier()

        @pl.when(cid == 0)
        def _():
            pltpu.sync_copy(acc_sh.at[pl.ds(row0, rpt)],
                            agg0_hbm.at[pl.ds(row0, rpt)])
            if with_deg:
                pltpu.sync_copy(deg_sh.at[pl.ds(row0, rpt)],
                                deg_hbm.at[pl.ds(row0, rpt)])

        @pl.when(cid == 1)
        def _():
            pltpu.sync_copy(acc_sh.at[pl.ds(row0, rpt)],
                            agg1_hbm.at[pl.ds(row0, rpt)])

    return seg_kernel(t0, t1, src, dst, z2, z1, ones_c)


def _dotT(a, w):
    return lax.dot_general(a, w, (((1,), (1,)), ((), ())),
                           preferred_element_type=jnp.float32)


def _tc_layer1(agg0, agg1, deg2, x, W1_l, W1_r, b1, W2_l, block_rows):
    n_rows, d_in = x.shape
    d_h = W1_l.shape[0]
    f_half = agg0.shape[1]
    grid = n_rows // block_rows

    def body(a0, a1, dg, xr, wl, wr, b, w2, h_ref, p0_ref, p1_ref):
        scale = 1.0 / jnp.maximum(dg[...], 1.0)
        wl_full = wl[...]
        s = _dotT(a0[...], wl_full[:, :f_half])
        s = s + _dotT(a1[...], wl_full[:, f_half:])
        h = jnp.maximum(s * scale + _dotT(xr[...], wr[...]) + b[...], 0.0)
        h_ref[...] = h
        p = _dotT(h, w2[...])
        p0_ref[...] = p[:, :f_half]
        p1_ref[...] = p[:, f_half:]

    return pl.pallas_call(
        body,
        grid=(grid,),
        in_specs=[
            pl.BlockSpec((block_rows, f_half), lambda i: (i, 0)),
            pl.BlockSpec((block_rows, f_half), lambda i: (i, 0)),
            pl.BlockSpec((block_rows, 1), lambda i: (i, 0)),
            pl.BlockSpec((block_rows, d_in), lambda i: (i, 0)),
            pl.BlockSpec((d_h, d_in), lambda i: (0, 0)),
            pl.BlockSpec((d_h, d_in), lambda i: (0, 0)),
            pl.BlockSpec((1, d_h), lambda i: (0, 0)),
            pl.BlockSpec((d_in, d_h), lambda i: (0, 0)),
        ],
        out_specs=[
            pl.BlockSpec((block_rows, d_h), lambda i: (i, 0)),
            pl.BlockSpec((block_rows, f_half), lambda i: (i, 0)),
            pl.BlockSpec((block_rows, f_half), lambda i: (i, 0)),
        ],
        out_shape=[
            jax.ShapeDtypeStruct((n_rows, d_h), jnp.float32),
            jax.ShapeDtypeStruct((n_rows, f_half), jnp.float32),
            jax.ShapeDtypeStruct((n_rows, f_half), jnp.float32),
        ],
    )(agg0, agg1, deg2, x, W1_l, W1_r, b1, W2_l)


def _tc_layer2(agg0, agg1, deg2, h, W2_r, b2, block_rows):
    n_rows, d_h = h.shape
    d_out = W2_r.shape[0]
    f_half = agg0.shape[1]
    grid = n_rows // block_rows

    def body(a0, a1, dg, hr, wr, b, out_ref):
        scale = 1.0 / jnp.maximum(dg[...], 1.0)
        m = _dotT(hr[...], wr[...]) + b[...]
        out_ref[:, :f_half] = a0[...] * scale + m[:, :f_half]
        out_ref[:, f_half:] = a1[...] * scale + m[:, f_half:]

    return pl.pallas_call(
        body,
        grid=(grid,),
        in_specs=[
            pl.BlockSpec((block_rows, f_half), lambda i: (i, 0)),
            pl.BlockSpec((block_rows, f_half), lambda i: (i, 0)),
            pl.BlockSpec((block_rows, 1), lambda i: (i, 0)),
            pl.BlockSpec((block_rows, d_h), lambda i: (i, 0)),
            pl.BlockSpec((d_out, d_h), lambda i: (0, 0)),
            pl.BlockSpec((1, d_out), lambda i: (0, 0)),
        ],
        out_specs=pl.BlockSpec((block_rows, d_out), lambda i: (i, 0)),
        out_shape=jax.ShapeDtypeStruct((n_rows, d_out), jnp.float32),
    )(agg0, agg1, deg2, h, W2_r, b2)


def kernel(x, edge_index, W1_l, b1, W1_r, W2_l, b2, W2_r):
    n, d_in = x.shape
    f_half = d_in // 2
    src = edge_index[0]
    dst = edge_index[1]
    n_pad = ((n + 128 * _NS - 1) // (128 * _NS)) * (128 * _NS)

    x0 = x[:, :f_half]
    x1 = x[:, f_half:]
    agg10, agg11, deg = _sc_segment_sum(x0, x1, src, dst, n_pad, True)
    deg2 = deg[:n].reshape(n, 1)
    h, p0, p1 = _tc_layer1(agg10[:n], agg11[:n], deg2, x,
                           W1_l, W1_r, b1.reshape(1, -1), W2_l,
                           block_rows=2000)
    agg20, agg21 = _sc_segment_sum(p0, p1, src, dst, n_pad, False)
    out = _tc_layer2(agg20[:n], agg21[:n], deg2, h,
                     W2_r, b2.reshape(1, -1), block_rows=2000)
    return out


# speedup vs baseline: 1.2114x; 1.0821x over previous
import functools

import jax
import jax.numpy as jnp
from jax import lax
from jax.experimental import pallas as pl
from jax.experimental.pallas import tpu as pltpu
from jax.experimental.pallas import tpu_sc as plsc

_NC = 2
_NS = 16
_C = 104
_CT = 16


def _sc_segment_sum(t0, t1, src, dst, n_pad, with_deg):
    n_rows, f_half = t0.shape
    e_total = src.shape[0]
    ept = e_total // _NS
    tails = (_CT,)
    n_main = (ept - sum(tails)) // _C
    assert n_main * _C + sum(tails) == ept and n_main % 12 == 0
    rpt = n_pad // _NS
    z2 = jnp.zeros((rpt, f_half), jnp.float32)
    z1 = jnp.zeros((rpt,), jnp.float32)
    ones_c = jnp.ones((_C,), jnp.float32)

    mesh = plsc.VectorSubcoreMesh(core_axis_name="c", subcore_axis_name="s",
                                  num_cores=_NC, num_subcores=_NS)

    out_type = [
        jax.ShapeDtypeStruct((n_pad, f_half), jnp.float32),
        jax.ShapeDtypeStruct((n_pad, f_half), jnp.float32),
    ]
    scratch = (
        [pltpu.VMEM((_C,), jnp.int32)] * 8
        + [pltpu.VMEM((_C, f_half), jnp.float32)] * 3
        + [
            pltpu.VMEM((_CT,), jnp.int32),
            pltpu.VMEM((_CT,), jnp.int32),
            pltpu.VMEM((_C,), jnp.float32),
            pltpu.VMEM_SHARED((n_pad, f_half), jnp.float32),
            pltpu.VMEM_SHARED((n_pad,), jnp.float32),
        ]
        + [pltpu.SemaphoreType.DMA] * 13
    )
    if with_deg:
        out_type.append(jax.ShapeDtypeStruct((n_pad,), jnp.float32))

    @functools.partial(pl.kernel, out_type=tuple(out_type), mesh=mesh,
                       scratch_types=scratch)
    def seg_kernel(t0_hbm, t1_hbm, src_hbm, dst_hbm, z2_hbm, z1_hbm, ones_hbm,
                   agg0_hbm, agg1_hbm, *rest):
        if with_deg:
            deg_hbm = rest[0]
            rest = rest[1:]
        (si0, di0, si1, di1, si2, di2, si3, di3, rows_v0, rows_v1, rows_v2,
         src_vt, dst_vt, ones_v, acc_sh, deg_sh,
         gsem0, gsem1, gsem2, isem0, isem1, isem2, isem3,
         ssem0, ssem1, ssem2, dsem0, dsem1, dsem2) = rest
        idx = ((si0, di0, isem0), (si1, di1, isem1),
               (si2, di2, isem2), (si3, di3, isem3))
        rows = ((rows_v0, gsem0), (rows_v1, gsem1), (rows_v2, gsem2))
        ssem = (ssem0, ssem1, ssem2)
        dsem = (dsem0, dsem1, dsem2)
        cid = lax.axis_index("c")
        sid = lax.axis_index("s")
        row0 = pl.multiple_of(sid * rpt, 8)
        ebase = pl.multiple_of(sid * ept, 8)

        pltpu.sync_copy(z2_hbm, acc_sh.at[pl.ds(row0, rpt)])

        if with_deg:
            @pl.when(cid == 0)
            def _():
                pltpu.sync_copy(z1_hbm, deg_sh.at[pl.ds(row0, rpt)])

        pltpu.sync_copy(ones_hbm, ones_v)
        plsc.subcore_barrier()

        def idx_off(k):
            return pl.multiple_of(ebase + k * _C, 8)

        def fire_idx(k, s):
            src_s, dst_s, isem_s = idx[s]
            off = idx_off(k)
            pltpu.async_copy(src_hbm.at[pl.ds(off, _C)], src_s, isem_s)
            pltpu.async_copy(dst_hbm.at[pl.ds(off, _C)], dst_s, isem_s)

        def wait_idx(k, s):
            src_s, dst_s, isem_s = idx[s]
            off = idx_off(k)
            pltpu.make_async_copy(src_hbm.at[pl.ds(off, _C)], src_s,
                                  isem_s).wait()
            pltpu.make_async_copy(dst_hbm.at[pl.ds(off, _C)], dst_s,
                                  isem_s).wait()

        def edge_loop(t_hbm, do_deg):
            for k in range(3):
                fire_idx(k, k)
            for k in range(2):
                wait_idx(k, k)
                pltpu.async_copy(t_hbm.at[idx[k][0]], rows[k][0], rows[k][1])

            def outer(g, carry):
                for j in range(12):
                    k = 12 * g + j
                    b = j % 3
                    rows_b, gsem_b = rows[b]
                    rows_n, gsem_n = rows[(b + 2) % 3]
                    dst_j = idx[j % 4][1]
                    dst_p = idx[(j + 3) % 4][1]

                    pltpu.make_async_copy(t_hbm.at[idx[j % 4][0]], rows_b,
                                          gsem_b).wait()
                    pltpu.async_copy(rows_b, acc_sh.at[dst_j], ssem[b],
                                     add=True)
                    if do_deg:
                        pltpu.async_copy(ones_v, deg_sh.at[dst_j], dsem[b],
                                         add=True)

                    @pl.when(k >= 1)
                    def _():
                        pltpu.make_async_copy(rows_n, acc_sh.at[dst_p],
                                              ssem[(b + 2) % 3]).wait()
                        if do_deg:
                            pltpu.make_async_copy(ones_v, deg_sh.at[dst_p],
                                                  dsem[(b + 2) % 3]).wait()

                    @pl.when(k + 3 < n_main)
                    def _():
                        fire_idx(k + 3, (j + 3) % 4)

                    @pl.when(k + 2 < n_main)
                    def _():
                        wait_idx(k + 2, (j + 2) % 4)
                        pltpu.async_copy(t_hbm.at[idx[(j + 2) % 4][0]],
                                         rows_n, gsem_n)
                return carry

            lax.fori_loop(0, n_main // 12, outer, 0)

            lb = (n_main - 1) % 3
            lj = (n_main - 1) % 4
            pltpu.make_async_copy(rows[lb][0], acc_sh.at[idx[lj][1]],
                                  ssem[lb]).wait()
            if do_deg:
                pltpu.make_async_copy(ones_v, deg_sh.at[idx[lj][1]],
                                      dsem[lb]).wait()

            o = pl.multiple_of(ebase + n_main * _C, 8)
            pltpu.sync_copy(src_hbm.at[pl.ds(o, _CT)], src_vt)
            pltpu.sync_copy(dst_hbm.at[pl.ds(o, _CT)], dst_vt)
            rows_t = rows_v0.at[pl.ds(0, _CT)]
            pltpu.async_copy(t_hbm.at[src_vt], rows_t, gsem0).wait()
            pltpu.sync_copy(rows_t, acc_sh.at[dst_vt], add=True)
            if do_deg:
                pltpu.sync_copy(ones_v.at[pl.ds(0, _CT)], deg_sh.at[dst_vt],
                                add=True)

        @pl.when(cid == 0)
        def _():
            edge_loop(t0_hbm, with_deg)

        @pl.when(cid == 1)
        def _():
            edge_loop(t1_hbm, False)

        plsc.subcore_barrier()

        @pl.when(cid == 0)
        def _():
            pltpu.sync_copy(acc_sh.at[pl.ds(row0, rpt)],
                            agg0_hbm.at[pl.ds(row0, rpt)])
            if with_deg:
                pltpu.sync_copy(deg_sh.at[pl.ds(row0, rpt)],
                                deg_hbm.at[pl.ds(row0, rpt)])

        @pl.when(cid == 1)
        def _():
            pltpu.sync_copy(acc_sh.at[pl.ds(row0, rpt)],
                            agg1_hbm.at[pl.ds(row0, rpt)])

    return seg_kernel(t0, t1, src, dst, z2, z1, ones_c)


def _dotT(a, w):
    return lax.dot_general(a, w, (((1,), (1,)), ((), ())),
                           preferred_element_type=jnp.float32)


def _tc_layer1(agg0, agg1, deg2, x, W1_l, W1_r, b1, W2_l, block_rows):
    n_rows, d_in = x.shape
    d_h = W1_l.shape[0]
    f_half = agg0.shape[1]
    grid = n_rows // block_rows

    def body(a0, a1, dg, xr, wl, wr, b, w2, h_ref, p0_ref, p1_ref):
        scale = 1.0 / jnp.maximum(dg[...], 1.0)
        wl_full = wl[...]
        s = _dotT(a0[...], wl_full[:, :f_half])
        s = s + _dotT(a1[...], wl_full[:, f_half:])
        h = jnp.maximum(s * scale + _dotT(xr[...], wr[...]) + b[...], 0.0)
        h_ref[...] = h
        p = _dotT(h, w2[...])
        p0_ref[...] = p[:, :f_half]
        p1_ref[...] = p[:, f_half:]

    return pl.pallas_call(
        body,
        grid=(grid,),
        in_specs=[
            pl.BlockSpec((block_rows, f_half), lambda i: (i, 0)),
            pl.BlockSpec((block_rows, f_half), lambda i: (i, 0)),
            pl.BlockSpec((block_rows, 1), lambda i: (i, 0)),
            pl.BlockSpec((block_rows, d_in), lambda i: (i, 0)),
            pl.BlockSpec((d_h, d_in), lambda i: (0, 0)),
            pl.BlockSpec((d_h, d_in), lambda i: (0, 0)),
            pl.BlockSpec((1, d_h), lambda i: (0, 0)),
            pl.BlockSpec((d_in, d_h), lambda i: (0, 0)),
        ],
        out_specs=[
            pl.BlockSpec((block_rows, d_h), lambda i: (i, 0)),
            pl.BlockSpec((block_rows, f_half), lambda i: (i, 0)),
            pl.BlockSpec((block_rows, f_half), lambda i: (i, 0)),
        ],
        out_shape=[
            jax.ShapeDtypeStruct((n_rows, d_h), jnp.float32),
            jax.ShapeDtypeStruct((n_rows, f_half), jnp.float32),
            jax.ShapeDtypeStruct((n_rows, f_half), jnp.float32),
        ],
    )(agg0, agg1, deg2, x, W1_l, W1_r, b1, W2_l)


def _tc_layer2(agg0, agg1, deg2, h, W2_r, b2, block_rows):
    n_rows, d_h = h.shape
    d_out = W2_r.shape[0]
    f_half = agg0.shape[1]
    grid = n_rows // block_rows

    def body(a0, a1, dg, hr, wr, b, out_ref):
        scale = 1.0 / jnp.maximum(dg[...], 1.0)
        m = _dotT(hr[...], wr[...]) + b[...]
        out_ref[:, :f_half] = a0[...] * scale + m[:, :f_half]
        out_ref[:, f_half:] = a1[...] * scale + m[:, f_half:]

    return pl.pallas_call(
        body,
        grid=(grid,),
        in_specs=[
            pl.BlockSpec((block_rows, f_half), lambda i: (i, 0)),
            pl.BlockSpec((block_rows, f_half), lambda i: (i, 0)),
            pl.BlockSpec((block_rows, 1), lambda i: (i, 0)),
            pl.BlockSpec((block_rows, d_h), lambda i: (i, 0)),
            pl.BlockSpec((d_out, d_h), lambda i: (0, 0)),
            pl.BlockSpec((1, d_out), lambda i: (0, 0)),
        ],
        out_specs=pl.BlockSpec((block_rows, d_out), lambda i: (i, 0)),
        out_shape=jax.ShapeDtypeStruct((n_rows, d_out), jnp.float32),
    )(agg0, agg1, deg2, h, W2_r, b2)


def kernel(x, edge_index, W1_l, b1, W1_r, W2_l, b2, W2_r):
    n, d_in = x.shape
    f_half = d_in // 2
    src = edge_index[0]
    dst = edge_index[1]
    n_pad = ((n + 128 * _NS - 1) // (128 * _NS)) * (128 * _NS)

    x0 = x[:, :f_half]
    x1 = x[:, f_half:]
    agg10, agg11, deg = _sc_segment_sum(x0, x1, src, dst, n_pad, True)
    deg2 = deg[:n].reshape(n, 1)
    h, p0, p1 = _tc_layer1(agg10[:n], agg11[:n], deg2, x,
                           W1_l, W1_r, b1.reshape(1, -1), W2_l,
                           block_rows=2000)
    agg20, agg21 = _sc_segment_sum(p0, p1, src, dst, n_pad, False)
    out = _tc_layer2(agg20[:n], agg21[:n], deg2, h,
                     W2_r, b2.reshape(1, -1), block_rows=2000)
    return out
